# baseline (device time: 48542 ns/iter reference)
import jax
import jax.numpy as jnp
from jax import lax
from jax.experimental import pallas as pl
from jax.experimental.pallas import tpu as pltpu

N_DEV = 32
EPS = 1e-5
N_TOT = 4096 * 128


def kernel(x, Wp):
    b, h, w, c = x.shape
    c2 = Wp.shape[1]

    def body(x_ref, wp_ref, out_ref, gather_buf, send_sems, recv_sems):
        my_pos = lax.axis_index("i")

        xs = x_ref[...]
        s = jnp.sum(xs, axis=(1, 2))
        sq = jnp.sum(xs * xs, axis=(1, 2))
        gather_buf[0] = jnp.concatenate([s, sq], axis=0)

        barrier_sem = pltpu.get_barrier_semaphore()
        for k in range(1, N_DEV):
            peer = lax.rem(my_pos + k, N_DEV)
            pl.semaphore_signal(
                barrier_sem, inc=1,
                device_id=(peer,), device_id_type=pl.DeviceIdType.MESH,
            )
        pl.semaphore_wait(barrier_sem, N_DEV - 1)

        rdmas = []
        for k in range(1, N_DEV):
            peer = lax.rem(my_pos + k, N_DEV)
            rdma = pltpu.make_async_remote_copy(
                src_ref=gather_buf.at[0],
                dst_ref=gather_buf.at[k],
                send_sem=send_sems.at[k],
                recv_sem=recv_sems.at[k],
                device_id=(peer,),
                device_id_type=pl.DeviceIdType.MESH,
            )
            rdma.start()
            rdmas.append(rdma)
        for rdma in rdmas:
            rdma.wait_send()
        for rdma in rdmas:
            rdma.wait_recv()

        tot = jnp.sum(gather_buf[...], axis=0)
        mean = tot[0:b] / N_TOT
        var = tot[b : 2 * b] / N_TOT - mean * mean
        rstd = lax.rsqrt(var + EPS)
        hn = ((xs - mean[:, None, None, :]) * rstd[:, None, None, :]).astype(
            jnp.bfloat16
        )
        a = hn * jax.nn.sigmoid(hn)
        a2 = a.reshape(b * h * w, c)
        wp = wp_ref[...].astype(jnp.bfloat16)
        out = jnp.dot(a2, wp, preferred_element_type=jnp.float32)
        out_ref[...] = out.reshape(b, h, w, c2).astype(jnp.bfloat16)

    return pl.pallas_call(
        body,
        out_shape=jax.ShapeDtypeStruct((b, h, w, c2), jnp.bfloat16),
        in_specs=[
            pl.BlockSpec(memory_space=pltpu.VMEM),
            pl.BlockSpec(memory_space=pltpu.VMEM),
        ],
        out_specs=pl.BlockSpec(memory_space=pltpu.VMEM),
        scratch_shapes=[
            pltpu.VMEM((N_DEV, 2 * b, c), jnp.float32),
            pltpu.SemaphoreType.DMA((N_DEV,)),
            pltpu.SemaphoreType.DMA((N_DEV,)),
        ],
        compiler_params=pltpu.CompilerParams(
            collective_id=0, vmem_limit_bytes=64 * 1024 * 1024
        ),
    )(x, Wp)


# device time: 29130 ns/iter; 1.6664x vs baseline; 1.6664x over previous
import jax
import jax.numpy as jnp
from jax import lax
from jax.experimental import pallas as pl
from jax.experimental.pallas import tpu as pltpu

N_DEV = 32
EPS = 1e-5
N_TOT = 4096 * 128


def kernel(x, Wp):
    xt = jnp.swapaxes(x, 2, 3)
    b, h, c, w = xt.shape
    c2 = Wp.shape[1]

    def body(x_ref, wp_ref, out_ref, gather_buf, send_sems, recv_sems):
        my_pos = lax.axis_index("i")

        xs = x_ref[...]
        s = jnp.sum(xs, axis=(1, 3))
        sq = jnp.sum(xs * xs, axis=(1, 3))
        gather_buf[0] = jnp.concatenate([s, sq], axis=0)

        barrier_sem = pltpu.get_barrier_semaphore()
        for k in range(1, N_DEV):
            peer = lax.rem(my_pos + k, N_DEV)
            pl.semaphore_signal(
                barrier_sem, inc=1,
                device_id=(peer,), device_id_type=pl.DeviceIdType.MESH,
            )
        pl.semaphore_wait(barrier_sem, N_DEV - 1)

        rdmas = []
        for k in range(1, N_DEV):
            peer = lax.rem(my_pos + k, N_DEV)
            rdma = pltpu.make_async_remote_copy(
                src_ref=gather_buf.at[0],
                dst_ref=gather_buf.at[k],
                send_sem=send_sems.at[k],
                recv_sem=recv_sems.at[k],
                device_id=(peer,),
                device_id_type=pl.DeviceIdType.MESH,
            )
            rdma.start()
            rdmas.append(rdma)
        for rdma in rdmas:
            rdma.wait_send()
        for rdma in rdmas:
            rdma.wait_recv()

        tot = jnp.sum(gather_buf[...], axis=0)
        mean = tot[0:b] / N_TOT
        var = tot[b : 2 * b] / N_TOT - mean * mean
        rstd = lax.rsqrt(var + EPS)
        hn = ((xs - mean[:, None, :, None]) * rstd[:, None, :, None]).astype(
            jnp.bfloat16
        )
        a = hn * jax.nn.sigmoid(hn)
        a3 = a.reshape(b * h, c, w)
        wp = wp_ref[...].astype(jnp.bfloat16)
        out = lax.dot_general(
            a3, wp,
            dimension_numbers=(((1,), (0,)), ((), ())),
            preferred_element_type=jnp.float32,
        )
        out_ref[...] = out.reshape(b, h, w, c2).astype(jnp.bfloat16)

    return pl.pallas_call(
        body,
        out_shape=jax.ShapeDtypeStruct((b, h, w, c2), jnp.bfloat16),
        in_specs=[
            pl.BlockSpec(memory_space=pltpu.VMEM),
            pl.BlockSpec(memory_space=pltpu.VMEM),
        ],
        out_specs=pl.BlockSpec(memory_space=pltpu.VMEM),
        scratch_shapes=[
            pltpu.VMEM((N_DEV, 2 * b, c), jnp.float32),
            pltpu.SemaphoreType.DMA((N_DEV,)),
            pltpu.SemaphoreType.DMA((N_DEV,)),
        ],
        compiler_params=pltpu.CompilerParams(
            collective_id=0, vmem_limit_bytes=64 * 1024 * 1024
        ),
    )(xt, Wp)
